# baseline (speedup 1.0000x reference)
import functools

import jax
import jax.numpy as jnp
from jax import lax
from jax.experimental import pallas as pl
from jax.experimental.pallas import tpu as pltpu
from jax.experimental.pallas import tpu_sc as plsc

_BATCH = 4
_SEQ = 4096
_HIDDEN = 1024
_NROWS = _BATCH * _SEQ
_NC = 2
_NS = 16
_NW = _NC * _NS
_POS_W = _SEQ // _NW
_PCHUNK = 8
_NCHUNK = _POS_W // _PCHUNK
_ROWS_C = _BATCH * _PCHUNK
_NBUF = 3
_LANES = 16
_COLS = _HIDDEN // _LANES
_CUNROLL = 8


def _emb_kernel(ids_hbm, word_hbm, pos_hbm, out_hbm, idx_v,
                wbuf0, wbuf1, wbuf2, pbuf0, pbuf1, pbuf2,
                gsem0, gsem1, gsem2, ssem0, ssem1, ssem2):
    wid = lax.axis_index("s") * _NC + lax.axis_index("c")
    p_base = wid * _POS_W
    wbufs, pbufs = (wbuf0, wbuf1, wbuf2), (pbuf0, pbuf1, pbuf2)
    gsems, ssems = (gsem0, gsem1, gsem2), (ssem0, ssem1, ssem2)

    pltpu.sync_copy(ids_hbm.at[pl.ds(wid * _NCHUNK, _NCHUNK)], idx_v)

    def in_copies(ci):
        par = ci % _NBUF
        pos0 = p_base + ci * _PCHUNK
        return [
            pltpu.make_async_copy(
                word_hbm.at[idx_v.at[ci]], wbufs[par], gsems[par]),
            pltpu.make_async_copy(
                pos_hbm.at[pl.ds(pos0, _PCHUNK)], pbufs[par], gsems[par]),
        ]

    def out_copies(ci):
        par = ci % _NBUF
        pos0 = p_base + ci * _PCHUNK
        return [
            pltpu.make_async_copy(
                wbufs[par].at[pl.ds(b * _PCHUNK, _PCHUNK)],
                out_hbm.at[pl.ds(b * _SEQ + pos0, _PCHUNK)], ssems[par])
            for b in range(_BATCH)
        ]

    for cp in in_copies(0) + in_copies(1):
        cp.start()

    for ci in range(_NCHUNK):
        if ci >= 1 and ci + 2 < _NCHUNK:
            for cp in out_copies(ci - 1):
                cp.wait()
        if ci + 2 < _NCHUNK:
            for cp in in_copies(ci + 2):
                cp.start()
        for cp in in_copies(ci):
            cp.wait()

        wb, pb = wbufs[ci % _NBUF], pbufs[ci % _NBUF]

        def add_row(p, c2, wb=wb, pb=pb):
            def add_cgroup(cg, c3):
                for cu in range(_CUNROLL):
                    sl = pl.ds(cg * _CUNROLL * _LANES + cu * _LANES, _LANES)
                    pv = pb[p, sl]
                    for b in range(_BATCH):
                        plsc.addupdate(wb.at[b * _PCHUNK + p, sl], pv)
                return c3
            return lax.fori_loop(0, _COLS // _CUNROLL, add_cgroup, c2)

        lax.fori_loop(0, _PCHUNK, add_row, 0)

        for cp in out_copies(ci):
            cp.start()

    for ci in range(_NCHUNK - 3, _NCHUNK):
        for cp in out_copies(ci):
            cp.wait()


@jax.jit
def _run(ids_perm, word_table, pos_table):
    mesh = plsc.VectorSubcoreMesh(core_axis_name="c", subcore_axis_name="s")
    f = functools.partial(
        pl.kernel,
        out_type=jax.ShapeDtypeStruct((_NROWS, _HIDDEN), jnp.float32),
        mesh=mesh,
        scratch_types=(
            [pltpu.VMEM((_NCHUNK, _ROWS_C), jnp.int32)]
            + [pltpu.VMEM((_ROWS_C, _HIDDEN), jnp.float32)] * _NBUF
            + [pltpu.VMEM((_PCHUNK, _HIDDEN), jnp.float32)] * _NBUF
            + [pltpu.SemaphoreType.DMA] * (2 * _NBUF)
        ),
    )(_emb_kernel)
    return f(ids_perm, word_table, pos_table)


def kernel(input_ids, word_table, pos_table):
    ids_perm = (input_ids.astype(jnp.int32)
                .reshape(_BATCH, _NW, _NCHUNK, _PCHUNK)
                .transpose(1, 2, 0, 3)
                .reshape(_NW * _NCHUNK, _ROWS_C))
    out = _run(ids_perm, word_table, pos_table)
    return out.reshape(_BATCH, _SEQ, _HIDDEN)


# speedup vs baseline: 1.0726x; 1.0726x over previous
import functools

import jax
import jax.numpy as jnp
from jax import lax
from jax.experimental import pallas as pl
from jax.experimental.pallas import tpu as pltpu
from jax.experimental.pallas import tpu_sc as plsc

_BATCH = 4
_SEQ = 4096
_HIDDEN = 1024
_NROWS = _BATCH * _SEQ
_NC = 2
_NS = 16
_NW = _NC * _NS
_POS_W = _SEQ // _NW
_PCHUNK = 8
_NCHUNK = _POS_W // _PCHUNK
_ROWS_C = _BATCH * _PCHUNK
_NBUF = 3
_LANES = 16
_COLS = _HIDDEN // _LANES
_CUNROLL = 8


def _emb_kernel(ids_hbm, word_hbm, pos_hbm, out_hbm, idx_v,
                wbuf0, wbuf1, wbuf2, pbuf0, pbuf1, pbuf2,
                gsem0, gsem1, gsem2, ssem0, ssem1, ssem2):
    wid = lax.axis_index("s") * _NC + lax.axis_index("c")
    p_base = wid * _POS_W
    wbufs, pbufs = (wbuf0, wbuf1, wbuf2), (pbuf0, pbuf1, pbuf2)
    gsems, ssems = (gsem0, gsem1, gsem2), (ssem0, ssem1, ssem2)

    pltpu.sync_copy(ids_hbm.at[pl.ds(wid * _NCHUNK, _NCHUNK)], idx_v)

    def in_copies(ci):
        par = ci % _NBUF
        pos0 = p_base + ci * _PCHUNK
        return [
            pltpu.make_async_copy(
                word_hbm.at[idx_v.at[ci]], wbufs[par], gsems[par]),
            pltpu.make_async_copy(
                pos_hbm.at[pl.ds(pos0, _PCHUNK)], pbufs[par], gsems[par]),
        ]

    def out_copies(ci):
        par = ci % _NBUF
        pos0 = p_base + ci * _PCHUNK
        return [
            pltpu.make_async_copy(
                wbufs[par].at[pl.ds(b * _PCHUNK, _PCHUNK)],
                out_hbm.at[pl.ds(b * _SEQ + pos0, _PCHUNK)], ssems[par])
            for b in range(_BATCH)
        ]

    for cp in in_copies(0) + in_copies(1):
        cp.start()

    for ci in range(_NCHUNK):
        if ci >= 1 and ci + 2 < _NCHUNK:
            for cp in out_copies(ci - 1):
                cp.wait()
        if ci + 2 < _NCHUNK:
            for cp in in_copies(ci + 2):
                cp.start()
        for cp in in_copies(ci):
            cp.wait()

        wb, pb = wbufs[ci % _NBUF], pbufs[ci % _NBUF]

        def add_half_row(h, c2, wb=wb, pb=pb):
            p = lax.div(h, 2)
            half = lax.rem(h, 2) * (_COLS // 2 * _LANES)
            for c in range(_COLS // 2):
                sl = pl.ds(half + c * _LANES, _LANES)
                pv = pb[p, sl]
                for b in range(_BATCH):
                    plsc.addupdate(wb.at[b * _PCHUNK + p, sl], pv)
            return c2

        lax.fori_loop(0, _PCHUNK * 2, add_half_row, 0)

        for cp in out_copies(ci):
            cp.start()

    for ci in range(_NCHUNK - 3, _NCHUNK):
        for cp in out_copies(ci):
            cp.wait()


@jax.jit
def _run(ids_perm, word_table, pos_table):
    mesh = plsc.VectorSubcoreMesh(core_axis_name="c", subcore_axis_name="s")
    f = functools.partial(
        pl.kernel,
        out_type=jax.ShapeDtypeStruct((_NROWS, _HIDDEN), jnp.float32),
        mesh=mesh,
        scratch_types=(
            [pltpu.VMEM((_NCHUNK, _ROWS_C), jnp.int32)]
            + [pltpu.VMEM((_ROWS_C, _HIDDEN), jnp.float32)] * _NBUF
            + [pltpu.VMEM((_PCHUNK, _HIDDEN), jnp.float32)] * _NBUF
            + [pltpu.SemaphoreType.DMA] * (2 * _NBUF)
        ),
    )(_emb_kernel)
    return f(ids_perm, word_table, pos_table)


def kernel(input_ids, word_table, pos_table):
    ids_perm = (input_ids.astype(jnp.int32)
                .reshape(_BATCH, _NW, _NCHUNK, _PCHUNK)
                .transpose(1, 2, 0, 3)
                .reshape(_NW * _NCHUNK, _ROWS_C))
    out = _run(ids_perm, word_table, pos_table)
    return out.reshape(_BATCH, _SEQ, _HIDDEN)


# speedup vs baseline: 1.1702x; 1.0910x over previous
import functools

import jax
import jax.numpy as jnp
from jax import lax
from jax.experimental import pallas as pl
from jax.experimental.pallas import tpu as pltpu
from jax.experimental.pallas import tpu_sc as plsc

_BATCH = 4
_SEQ = 4096
_HIDDEN = 1024
_NROWS = _BATCH * _SEQ
_NC = 2
_NS = 16
_NW = _NC * _NS
_POS_W = _SEQ // _NW
_PCHUNK = 8
_NCHUNK = _POS_W // _PCHUNK
_ROWS_C = _BATCH * _PCHUNK
_NBUF = 3
_NTRIPLE = (_NCHUNK - 1) // _NBUF
_LANES = 16
_COLS = _HIDDEN // _LANES


def _emb_kernel(ids_hbm, word_hbm, pos_hbm, out_hbm, idx_v,
                wbuf0, wbuf1, wbuf2, pbuf0, pbuf1, pbuf2,
                gsem0, gsem1, gsem2, ssem0, ssem1, ssem2):
    wid = lax.axis_index("s") * _NC + lax.axis_index("c")
    p_base = wid * _POS_W
    wbufs, pbufs = (wbuf0, wbuf1, wbuf2), (pbuf0, pbuf1, pbuf2)
    gsems, ssems = (gsem0, gsem1, gsem2), (ssem0, ssem1, ssem2)

    pltpu.sync_copy(ids_hbm.at[pl.ds(wid * _NCHUNK, _NCHUNK)], idx_v)

    def in_copies(ci, par):
        pos0 = p_base + ci * _PCHUNK
        return [
            pltpu.make_async_copy(
                word_hbm.at[idx_v.at[ci]], wbufs[par], gsems[par]),
            pltpu.make_async_copy(
                pos_hbm.at[pl.ds(pos0, _PCHUNK)], pbufs[par], gsems[par]),
        ]

    def out_copies(ci, par):
        pos0 = p_base + ci * _PCHUNK
        return [
            pltpu.make_async_copy(
                wbufs[par].at[pl.ds(b * _PCHUNK, _PCHUNK)],
                out_hbm.at[pl.ds(b * _SEQ + pos0, _PCHUNK)], ssems[par])
            for b in range(_BATCH)
        ]

    def accumulate(par):
        wb, pb = wbufs[par], pbufs[par]

        def add_row(p, c2):
            for c in range(_COLS):
                sl = pl.ds(c * _LANES, _LANES)
                pv = pb[p, sl]
                for b in range(_BATCH):
                    plsc.addupdate(wb.at[b * _PCHUNK + p, sl], pv)
            return c2

        lax.fori_loop(0, _PCHUNK, add_row, 0)

    for cp in in_copies(0, 0) + in_copies(1, 1):
        cp.start()

    def triple_body(q, carry):
        for sub in range(_NBUF):
            ci = q * _NBUF + sub
            par = sub
            nxt = (sub + 2) % _NBUF
            if sub == 0:
                @pl.when(q > 0)
                def _():
                    for cp in out_copies(ci - 1, nxt):
                        cp.wait()
            else:
                for cp in out_copies(ci - 1, par - 1):
                    cp.wait()
            if sub == _NBUF - 1:
                @pl.when(q < _NTRIPLE - 1)
                def _():
                    for cp in in_copies(ci + 2, nxt):
                        cp.start()
            else:
                for cp in in_copies(ci + 2, nxt):
                    cp.start()
            for cp in in_copies(ci, par):
                cp.wait()
            accumulate(par)
            for cp in out_copies(ci, par):
                cp.start()
        return carry

    lax.fori_loop(0, _NTRIPLE, triple_body, 0)

    ci = _NCHUNK - 1
    for cp in out_copies(ci - 1, 2):
        cp.wait()
    for cp in in_copies(ci, 0):
        cp.wait()
    accumulate(0)
    for cp in out_copies(ci, 0):
        cp.start()
        cp.wait()


@jax.jit
def _run(ids_perm, word_table, pos_table):
    mesh = plsc.VectorSubcoreMesh(core_axis_name="c", subcore_axis_name="s")
    f = functools.partial(
        pl.kernel,
        out_type=jax.ShapeDtypeStruct((_NROWS, _HIDDEN), jnp.float32),
        mesh=mesh,
        scratch_types=(
            [pltpu.VMEM((_NCHUNK, _ROWS_C), jnp.int32)]
            + [pltpu.VMEM((_ROWS_C, _HIDDEN), jnp.float32)] * _NBUF
            + [pltpu.VMEM((_PCHUNK, _HIDDEN), jnp.float32)] * _NBUF
            + [pltpu.SemaphoreType.DMA] * (2 * _NBUF)
        ),
    )(_emb_kernel)
    return f(ids_perm, word_table, pos_table)


def kernel(input_ids, word_table, pos_table):
    ids_perm = (input_ids.astype(jnp.int32)
                .reshape(_BATCH, _NW, _NCHUNK, _PCHUNK)
                .transpose(1, 2, 0, 3)
                .reshape(_NW * _NCHUNK, _ROWS_C))
    out = _run(ids_perm, word_table, pos_table)
    return out.reshape(_BATCH, _SEQ, _HIDDEN)


# speedup vs baseline: 1.2250x; 1.0468x over previous
import functools

import jax
import jax.numpy as jnp
from jax import lax
from jax.experimental import pallas as pl
from jax.experimental.pallas import tpu as pltpu
from jax.experimental.pallas import tpu_sc as plsc

_BATCH = 4
_SEQ = 4096
_HIDDEN = 1024
_NROWS = _BATCH * _SEQ
_NC = 2
_NS = 16
_NW = _NC * _NS
_POS_W = _SEQ // _NW
_PCHUNK = 8
_NCHUNK = _POS_W // _PCHUNK
_NPAIR = _NCHUNK // 2
_ROWS_C = _BATCH * _PCHUNK
_LANES = 16
_COLS = _HIDDEN // _LANES


def _emb_kernel(ids_hbm, word_hbm, pos_hbm, out_hbm, idx_v,
                wbuf0, wbuf1, pbuf0, pbuf1, gsem0, gsem1, ssem0, ssem1):
    wid = lax.axis_index("s") * _NC + lax.axis_index("c")
    p_base = wid * _POS_W
    wbufs, pbufs = (wbuf0, wbuf1), (pbuf0, pbuf1)
    gsems, ssems = (gsem0, gsem1), (ssem0, ssem1)

    pltpu.sync_copy(ids_hbm.at[pl.ds(wid * _NCHUNK, _NCHUNK)], idx_v)

    def in_copies(ci, par):
        pos0 = p_base + ci * _PCHUNK
        return [
            pltpu.make_async_copy(
                word_hbm.at[idx_v.at[ci]], wbufs[par], gsems[par]),
            pltpu.make_async_copy(
                pos_hbm.at[pl.ds(pos0, _PCHUNK)], pbufs[par], gsems[par]),
        ]

    def out_copies(ci, par):
        pos0 = p_base + ci * _PCHUNK
        return [
            pltpu.make_async_copy(
                wbufs[par].at[pl.ds(b * _PCHUNK, _PCHUNK)],
                out_hbm.at[pl.ds(b * _SEQ + pos0, _PCHUNK)], ssems[par])
            for b in range(_BATCH)
        ]

    for cp in in_copies(0, 0):
        cp.start()

    def pair_body(ci2, carry):
        for par in range(2):
            ci = ci2 * 2 + par
            if par == 1:
                for cp in out_copies(ci - 1, 0):
                    cp.wait()
            else:
                @pl.when(ci2 > 0)
                def _():
                    for cp in out_copies(ci - 1, 1):
                        cp.wait()
            if par == 0:
                for cp in in_copies(ci + 1, 1):
                    cp.start()
            else:
                @pl.when(ci2 < _NPAIR - 1)
                def _():
                    for cp in in_copies(ci + 1, 0):
                        cp.start()
            for cp in in_copies(ci, par):
                cp.wait()

            wb, pb = wbufs[par], pbufs[par]

            def add_row(p, c2):
                for c in range(_COLS):
                    sl = pl.ds(c * _LANES, _LANES)
                    pv = pb[p, sl]
                    for b in range(_BATCH):
                        plsc.addupdate(wb.at[b * _PCHUNK + p, sl], pv)
                return c2

            lax.fori_loop(0, _PCHUNK, add_row, 0)

            for cp in out_copies(ci, par):
                cp.start()
        return carry

    lax.fori_loop(0, _NPAIR, pair_body, 0)
    for cp in out_copies(_NCHUNK - 1, 1):
        cp.wait()


@jax.jit
def _run(ids_perm, word_table, pos_table):
    mesh = plsc.VectorSubcoreMesh(core_axis_name="c", subcore_axis_name="s")
    f = functools.partial(
        pl.kernel,
        out_type=jax.ShapeDtypeStruct((_NROWS, _HIDDEN), jnp.float32),
        mesh=mesh,
        scratch_types=[
            pltpu.VMEM((_NCHUNK, _ROWS_C), jnp.int32),
            pltpu.VMEM((_ROWS_C, _HIDDEN), jnp.float32),
            pltpu.VMEM((_ROWS_C, _HIDDEN), jnp.float32),
            pltpu.VMEM((_PCHUNK, _HIDDEN), jnp.float32),
            pltpu.VMEM((_PCHUNK, _HIDDEN), jnp.float32),
            pltpu.SemaphoreType.DMA,
            pltpu.SemaphoreType.DMA,
            pltpu.SemaphoreType.DMA,
            pltpu.SemaphoreType.DMA,
        ],
    )(_emb_kernel)
    return f(ids_perm, word_table, pos_table)


def kernel(input_ids, word_table, pos_table):
    ids_perm = (input_ids.astype(jnp.int32)
                .reshape(_BATCH, _NW, _NCHUNK, _PCHUNK)
                .transpose(1, 2, 0, 3)
                .reshape(_NW * _NCHUNK, _ROWS_C))
    out = _run(ids_perm, word_table, pos_table)
    return out.reshape(_BATCH, _SEQ, _HIDDEN)
